# baseline (speedup 1.0000x reference)
import functools

import jax
import jax.numpy as jnp
from jax.experimental import pallas as pl
from jax.experimental.pallas import tpu as pltpu


def _moe_body(ns, be_ref, na_ref, x_ref, tok_ref, w_ref, w1_ref, b1_ref,
              w2_ref, b2_ref, out_ref, xs_ref, ys_ref):
    BLK, S, NF = ns
    b = pl.program_id(0)
    f = pl.program_id(1)
    active = b < na_ref[0]

    tok = tok_ref[0]
    iota = jax.lax.broadcasted_iota(jnp.int32, (BLK, S), 1)

    @pl.when(active & (f == 0))
    def _gather():
        onehot = (tok == iota).astype(jnp.bfloat16)
        xs_ref[...] = jnp.dot(onehot, x_ref[...],
                              preferred_element_type=jnp.float32
                              ).astype(jnp.bfloat16)
        ys_ref[...] = jnp.broadcast_to(b2_ref[0], ys_ref.shape)

    @pl.when(active)
    def _ffn():
        h = jnp.dot(xs_ref[...], w1_ref[0].astype(jnp.bfloat16),
                    preferred_element_type=jnp.float32) + b1_ref[0]
        h = jax.nn.gelu(h, approximate=True)
        ys_ref[...] += jnp.dot(h.astype(jnp.bfloat16),
                               w2_ref[0].astype(jnp.bfloat16),
                               preferred_element_type=jnp.float32)

    @pl.when(f == NF - 1)
    def _scatter():
        @pl.when(b == 0)
        def _init():
            out_ref[...] = jnp.zeros_like(out_ref)

        @pl.when(active)
        def _add():
            oh_w = jnp.where(tok == iota, w_ref[0], 0.0).astype(jnp.bfloat16)
            out_ref[...] += jax.lax.dot_general(
                oh_w, ys_ref[...].astype(jnp.bfloat16),
                (((0,), (0,)), ((), ())),
                preferred_element_type=jnp.float32)


def kernel(hidden_states, router_w, router_b, fc1_w, fc1_b, fc2_w, fc2_b):
    BATCH, S, D = hidden_states.shape
    E = router_w.shape[1]
    F = fc1_w.shape[2]
    K = 2
    T = BATCH * S
    BLK = 512 if T % 512 == 0 else 8
    FT = 512 if F % 512 == 0 else F
    NF = F // FT
    NB = (T * K) // BLK + E

    x = hidden_states.reshape(T, D)

    logits = x @ router_w + router_b
    probs = jax.nn.softmax(logits, axis=-1)
    topw, topi = jax.lax.top_k(probs, K)
    topw = topw / jnp.sum(topw, axis=-1, keepdims=True)

    e_flat = topi.reshape(-1)
    w_flat = topw.reshape(-1)
    tok_flat = jnp.arange(T * K, dtype=jnp.int32) // K
    onehot_e = (e_flat[:, None] == jnp.arange(E)[None, :]).astype(jnp.int32)
    cum = jnp.cumsum(onehot_e, axis=0)
    rank = jnp.take_along_axis(cum, e_flat[:, None], axis=1)[:, 0] - 1
    counts = cum[-1]
    nblk_e = (counts + BLK - 1) // BLK
    blk_start = jnp.concatenate([jnp.zeros((1,), jnp.int32),
                                 jnp.cumsum(nblk_e)[:-1].astype(jnp.int32)])
    row_start = BLK * blk_start
    dest = row_start[e_flat] + rank
    na = jnp.sum(nblk_e).astype(jnp.int32)

    bounds = jnp.cumsum(nblk_e)
    barange = jnp.arange(NB, dtype=jnp.int32)
    be_raw = jnp.minimum(
        jnp.sum(barange[:, None] >= bounds[None, :], axis=1), E - 1
    ).astype(jnp.int32)
    be_last = be_raw[jnp.maximum(na - 1, 0)]
    block_expert = jnp.where(barange < na, be_raw, be_last)

    P = NB * BLK
    tok_col = jnp.zeros((P,), jnp.int32).at[dest].set(
        tok_flat.astype(jnp.int32), unique_indices=True).reshape(NB, BLK, 1)
    w_col = jnp.zeros((P,), jnp.float32).at[dest].set(
        w_flat, unique_indices=True).reshape(NB, BLK, 1)

    fc1_b3 = fc1_b.reshape(E, 1, F)
    fc2_b3 = fc2_b.reshape(E, 1, D)

    def wmap(b, f, be, na_s):
        ff = jnp.where(b < na_s[0], f, NF - 1)
        return (be[b], 0, ff)

    grid_spec = pltpu.PrefetchScalarGridSpec(
        num_scalar_prefetch=2,
        grid=(NB, NF),
        in_specs=[
            pl.BlockSpec((T, D), lambda b, f, be, na_s: (0, 0)),
            pl.BlockSpec((1, BLK, 1), lambda b, f, be, na_s: (b, 0, 0)),
            pl.BlockSpec((1, BLK, 1), lambda b, f, be, na_s: (b, 0, 0)),
            pl.BlockSpec((1, D, FT), wmap),
            pl.BlockSpec((1, 1, FT),
                         lambda b, f, be, na_s:
                         (be[b], 0, jnp.where(b < na_s[0], f, NF - 1))),
            pl.BlockSpec((1, FT, D),
                         lambda b, f, be, na_s:
                         (be[b], jnp.where(b < na_s[0], f, NF - 1), 0)),
            pl.BlockSpec((1, 1, D), lambda b, f, be, na_s: (be[b], 0, 0)),
        ],
        out_specs=pl.BlockSpec((T, D), lambda b, f, be, na_s: (0, 0)),
        scratch_shapes=[
            pltpu.VMEM((BLK, D), jnp.bfloat16),
            pltpu.VMEM((BLK, D), jnp.float32),
        ],
    )

    out = pl.pallas_call(
        functools.partial(_moe_body, (BLK, T, NF)),
        grid_spec=grid_spec,
        out_shape=jax.ShapeDtypeStruct((T, D), jnp.float32),
        compiler_params=pltpu.CompilerParams(
            dimension_semantics=("arbitrary", "arbitrary")),
    )(block_expert, jnp.full((1,), na, jnp.int32),
      x.astype(jnp.bfloat16), tok_col, w_col, fc1_w, fc1_b3, fc2_w, fc2_b3)

    return out.reshape(BATCH, S, D)


# speedup vs baseline: 1.0293x; 1.0293x over previous
import functools

import jax
import jax.numpy as jnp
from jax.experimental import pallas as pl
from jax.experimental.pallas import tpu as pltpu


def _moe_body(ns, be_ref, na_ref, x_ref, tok_ref, w_ref, w1_ref, b1_ref,
              w2_ref, b2_ref, out_ref, xs_ref, ys_ref):
    BLK, S, NF = ns
    b = pl.program_id(0)
    f = pl.program_id(1)
    active = b < na_ref[0]

    tok = tok_ref[0]
    iota = jax.lax.broadcasted_iota(jnp.int32, (BLK, S), 1)

    @pl.when(active & (f == 0))
    def _gather():
        onehot = (tok == iota).astype(jnp.bfloat16)
        xs_ref[...] = jnp.dot(onehot, x_ref[...],
                              preferred_element_type=jnp.float32
                              ).astype(jnp.bfloat16)
        ys_ref[...] = jnp.broadcast_to(b2_ref[0], ys_ref.shape)

    @pl.when(active)
    def _ffn():
        h = jnp.dot(xs_ref[...], w1_ref[0].astype(jnp.bfloat16),
                    preferred_element_type=jnp.float32) + b1_ref[0]
        h = jax.nn.gelu(h, approximate=True)
        ys_ref[...] += jnp.dot(h.astype(jnp.bfloat16),
                               w2_ref[0].astype(jnp.bfloat16),
                               preferred_element_type=jnp.float32)

    @pl.when(f == NF - 1)
    def _scatter():
        @pl.when(b == 0)
        def _init():
            out_ref[...] = jnp.zeros_like(out_ref)

        @pl.when(active)
        def _add():
            oh_w = jnp.where(tok == iota, w_ref[0], 0.0).astype(jnp.bfloat16)
            out_ref[...] += jax.lax.dot_general(
                oh_w, ys_ref[...].astype(jnp.bfloat16),
                (((0,), (0,)), ((), ())),
                preferred_element_type=jnp.float32)


def kernel(hidden_states, router_w, router_b, fc1_w, fc1_b, fc2_w, fc2_b):
    BATCH, S, D = hidden_states.shape
    E = router_w.shape[1]
    F = fc1_w.shape[2]
    K = 2
    T = BATCH * S
    BLK = 512 if T % 512 == 0 else 8
    FT = 512 if F % 512 == 0 else F
    NF = F // FT
    NB = (T * K) // BLK + E

    x = hidden_states.reshape(T, D)

    logits = x @ router_w + router_b
    probs = jax.nn.softmax(logits, axis=-1)
    i1 = jnp.argmax(probs, axis=-1).astype(jnp.int32)
    m1 = jnp.max(probs, axis=-1)
    erange = jnp.arange(E, dtype=jnp.int32)
    masked = jnp.where(i1[:, None] == erange[None, :], -jnp.inf, probs)
    i2 = jnp.argmax(masked, axis=-1).astype(jnp.int32)
    m2 = jnp.max(masked, axis=-1)
    denom = m1 + m2
    topi = jnp.stack([i1, i2], axis=-1)
    topw = jnp.stack([m1 / denom, m2 / denom], axis=-1)

    e_flat = topi.reshape(-1)
    w_flat = topw.reshape(-1)
    tok_flat = jnp.arange(T * K, dtype=jnp.int32) // K
    onehot_e = (e_flat[:, None] == erange[None, :]).astype(jnp.int32)
    cum = jnp.cumsum(onehot_e, axis=0)
    rank = jnp.sum(cum * onehot_e, axis=1) - 1
    counts = cum[-1]
    nblk_e = (counts + BLK - 1) // BLK
    blk_start = jnp.concatenate([jnp.zeros((1,), jnp.int32),
                                 jnp.cumsum(nblk_e)[:-1].astype(jnp.int32)])
    row_start = BLK * blk_start
    dest = jnp.sum(onehot_e * row_start[None, :], axis=1) + rank
    na = jnp.sum(nblk_e).astype(jnp.int32)

    bounds = jnp.cumsum(nblk_e)
    barange = jnp.arange(NB, dtype=jnp.int32)
    be_raw = jnp.minimum(
        jnp.sum(barange[:, None] >= bounds[None, :], axis=1), E - 1
    ).astype(jnp.int32)
    be_last = be_raw[jnp.maximum(na - 1, 0)]
    block_expert = jnp.where(barange < na, be_raw, be_last)

    P = NB * BLK
    tok_col = jnp.zeros((P,), jnp.int32).at[dest].set(
        tok_flat.astype(jnp.int32), unique_indices=True).reshape(NB, BLK, 1)
    w_col = jnp.zeros((P,), jnp.float32).at[dest].set(
        w_flat, unique_indices=True).reshape(NB, BLK, 1)

    fc1_b3 = fc1_b.reshape(E, 1, F)
    fc2_b3 = fc2_b.reshape(E, 1, D)

    def wmap(b, f, be, na_s):
        ff = jnp.where(b < na_s[0], f, NF - 1)
        return (be[b], 0, ff)

    grid_spec = pltpu.PrefetchScalarGridSpec(
        num_scalar_prefetch=2,
        grid=(NB, NF),
        in_specs=[
            pl.BlockSpec((T, D), lambda b, f, be, na_s: (0, 0)),
            pl.BlockSpec((1, BLK, 1), lambda b, f, be, na_s: (b, 0, 0)),
            pl.BlockSpec((1, BLK, 1), lambda b, f, be, na_s: (b, 0, 0)),
            pl.BlockSpec((1, D, FT), wmap),
            pl.BlockSpec((1, 1, FT),
                         lambda b, f, be, na_s:
                         (be[b], 0, jnp.where(b < na_s[0], f, NF - 1))),
            pl.BlockSpec((1, FT, D),
                         lambda b, f, be, na_s:
                         (be[b], jnp.where(b < na_s[0], f, NF - 1), 0)),
            pl.BlockSpec((1, 1, D), lambda b, f, be, na_s: (be[b], 0, 0)),
        ],
        out_specs=pl.BlockSpec((T, D), lambda b, f, be, na_s: (0, 0)),
        scratch_shapes=[
            pltpu.VMEM((BLK, D), jnp.bfloat16),
            pltpu.VMEM((BLK, D), jnp.float32),
        ],
    )

    out = pl.pallas_call(
        functools.partial(_moe_body, (BLK, T, NF)),
        grid_spec=grid_spec,
        out_shape=jax.ShapeDtypeStruct((T, D), jnp.float32),
        compiler_params=pltpu.CompilerParams(
            dimension_semantics=("arbitrary", "arbitrary")),
    )(block_expert, jnp.full((1,), na, jnp.int32),
      x.astype(jnp.bfloat16), tok_col, w_col, fc1_w, fc1_b3, fc2_w, fc2_b3)

    return out.reshape(BATCH, S, D)


# speedup vs baseline: 1.0823x; 1.0515x over previous
import functools

import jax
import jax.numpy as jnp
from jax.experimental import pallas as pl
from jax.experimental.pallas import tpu as pltpu


def _moe_body(ns, be_ref, na_ref, x_ref, d0_ref, d1_ref, w0_ref, w1r_ref,
              w1_ref, b1_ref, w2_ref, b2_ref, out_ref, xs_ref, ys_ref):
    BLK, S, NF = ns
    b = pl.program_id(0)
    f = pl.program_id(1)
    active = b < na_ref[0]

    slot = b * BLK + jax.lax.broadcasted_iota(jnp.int32, (BLK, 1), 0)

    @pl.when(active & (f == 0))
    def _gather():
        mask0 = slot == d0_ref[...]
        mask1 = slot == d1_ref[...]
        onehot = (mask0 | mask1).astype(jnp.bfloat16)
        xs_ref[...] = jnp.dot(onehot, x_ref[...],
                              preferred_element_type=jnp.float32
                              ).astype(jnp.bfloat16)
        ys_ref[...] = jnp.broadcast_to(b2_ref[0], ys_ref.shape)

    @pl.when(active)
    def _ffn():
        h = jnp.dot(xs_ref[...], w1_ref[0].astype(jnp.bfloat16),
                    preferred_element_type=jnp.float32) + b1_ref[0]
        h = jax.nn.gelu(h, approximate=True)
        ys_ref[...] += jnp.dot(h.astype(jnp.bfloat16),
                               w2_ref[0].astype(jnp.bfloat16),
                               preferred_element_type=jnp.float32)

    @pl.when(f == NF - 1)
    def _scatter():
        @pl.when(b == 0)
        def _init():
            out_ref[...] = jnp.zeros_like(out_ref)

        @pl.when(active)
        def _add():
            mask0 = slot == d0_ref[...]
            mask1 = slot == d1_ref[...]
            oh_w = (jnp.where(mask0, w0_ref[...], 0.0)
                    + jnp.where(mask1, w1r_ref[...], 0.0)).astype(jnp.bfloat16)
            out_ref[...] += jax.lax.dot_general(
                oh_w, ys_ref[...].astype(jnp.bfloat16),
                (((0,), (0,)), ((), ())),
                preferred_element_type=jnp.float32)


def kernel(hidden_states, router_w, router_b, fc1_w, fc1_b, fc2_w, fc2_b):
    BATCH, S, D = hidden_states.shape
    E = router_w.shape[1]
    F = fc1_w.shape[2]
    K = 2
    T = BATCH * S
    BLK = 512 if T % 512 == 0 else 8
    FT = 512 if F % 512 == 0 else F
    NF = F // FT
    NB = (T * K) // BLK + E

    x = hidden_states.reshape(T, D)

    logits = x @ router_w + router_b
    probs = jax.nn.softmax(logits, axis=-1)
    i1 = jnp.argmax(probs, axis=-1).astype(jnp.int32)
    m1 = jnp.max(probs, axis=-1)
    erange = jnp.arange(E, dtype=jnp.int32)
    masked = jnp.where(i1[:, None] == erange[None, :], -jnp.inf, probs)
    i2 = jnp.argmax(masked, axis=-1).astype(jnp.int32)
    m2 = jnp.max(masked, axis=-1)
    denom = m1 + m2

    topi = jnp.stack([i1, i2], axis=-1)
    e_flat = topi.reshape(-1)
    onehot_e = (e_flat[:, None] == erange[None, :]).astype(jnp.int32)
    cum = jnp.cumsum(onehot_e, axis=0)
    rank = jnp.sum(cum * onehot_e, axis=1) - 1
    counts = cum[-1]
    nblk_e = (counts + BLK - 1) // BLK
    blk_start = jnp.concatenate([jnp.zeros((1,), jnp.int32),
                                 jnp.cumsum(nblk_e)[:-1].astype(jnp.int32)])
    row_start = BLK * blk_start
    dest = jnp.sum(onehot_e * row_start[None, :], axis=1) + rank
    na = jnp.sum(nblk_e).astype(jnp.int32)

    bounds = jnp.cumsum(nblk_e)
    barange = jnp.arange(NB, dtype=jnp.int32)
    be_raw = jnp.minimum(
        jnp.sum(barange[:, None] >= bounds[None, :], axis=1), E - 1
    ).astype(jnp.int32)
    be_last = be_raw[jnp.maximum(na - 1, 0)]
    block_expert = jnp.where(barange < na, be_raw, be_last)

    dest2 = dest.reshape(T, K)
    d0 = dest2[:, 0].reshape(1, T)
    d1 = dest2[:, 1].reshape(1, T)
    w0 = (m1 / denom).reshape(1, T)
    w1r = (m2 / denom).reshape(1, T)

    fc1_b3 = fc1_b.reshape(E, 1, F)
    fc2_b3 = fc2_b.reshape(E, 1, D)

    def wmap(b, f, be, na_s):
        ff = jnp.where(b < na_s[0], f, NF - 1)
        return (be[b], 0, ff)

    _res = lambda b, f, be, na_s: (0, 0)
    grid_spec = pltpu.PrefetchScalarGridSpec(
        num_scalar_prefetch=2,
        grid=(NB, NF),
        in_specs=[
            pl.BlockSpec((T, D), _res),
            pl.BlockSpec((1, T), _res),
            pl.BlockSpec((1, T), _res),
            pl.BlockSpec((1, T), _res),
            pl.BlockSpec((1, T), _res),
            pl.BlockSpec((1, D, FT), wmap),
            pl.BlockSpec((1, 1, FT),
                         lambda b, f, be, na_s:
                         (be[b], 0, jnp.where(b < na_s[0], f, NF - 1))),
            pl.BlockSpec((1, FT, D),
                         lambda b, f, be, na_s:
                         (be[b], jnp.where(b < na_s[0], f, NF - 1), 0)),
            pl.BlockSpec((1, 1, D), lambda b, f, be, na_s: (be[b], 0, 0)),
        ],
        out_specs=pl.BlockSpec((T, D), _res),
        scratch_shapes=[
            pltpu.VMEM((BLK, D), jnp.bfloat16),
            pltpu.VMEM((BLK, D), jnp.float32),
        ],
    )

    out = pl.pallas_call(
        functools.partial(_moe_body, (BLK, T, NF)),
        grid_spec=grid_spec,
        out_shape=jax.ShapeDtypeStruct((T, D), jnp.float32),
        compiler_params=pltpu.CompilerParams(
            dimension_semantics=("arbitrary", "arbitrary")),
    )(block_expert, jnp.full((1,), na, jnp.int32),
      x.astype(jnp.bfloat16), d0, d1, w0, w1r, fc1_w, fc1_b3, fc2_w, fc2_b3)

    return out.reshape(BATCH, S, D)


# speedup vs baseline: 1.0970x; 1.0136x over previous
import functools

import jax
import jax.numpy as jnp
from jax.experimental import pallas as pl
from jax.experimental.pallas import tpu as pltpu


def _moe_body(ns, be_ref, na_ref, x_ref, d0_ref, d1_ref, w0_ref, w1r_ref,
              w1_ref, b1_ref, w2_ref, b2_ref, out_ref, xs_ref, ys_ref):
    BLK, S, NF = ns
    b = pl.program_id(0)
    f = pl.program_id(1)
    active = b < na_ref[0]

    slot = b * BLK + jax.lax.broadcasted_iota(jnp.int32, (BLK, 1), 0)

    @pl.when(active & (f == 0))
    def _gather():
        mask0 = slot == d0_ref[...]
        mask1 = slot == d1_ref[...]
        onehot = (mask0 | mask1).astype(jnp.bfloat16)
        xs_ref[...] = jnp.dot(onehot, x_ref[...],
                              preferred_element_type=jnp.float32
                              ).astype(jnp.bfloat16)
        ys_ref[...] = jnp.broadcast_to(b2_ref[0], ys_ref.shape)

    @pl.when(active)
    def _ffn():
        h = jnp.dot(xs_ref[...], w1_ref[0].astype(jnp.bfloat16),
                    preferred_element_type=jnp.float32) + b1_ref[0]
        h = jax.nn.gelu(h, approximate=True)
        ys_ref[...] += jnp.dot(h.astype(jnp.bfloat16),
                               w2_ref[0].astype(jnp.bfloat16),
                               preferred_element_type=jnp.float32)

    @pl.when(f == NF - 1)
    def _scatter():
        @pl.when(b == 0)
        def _init():
            out_ref[...] = jnp.zeros_like(out_ref)

        @pl.when(active)
        def _add():
            mask0 = slot == d0_ref[...]
            mask1 = slot == d1_ref[...]
            oh_w = (jnp.where(mask0, w0_ref[...], 0.0)
                    + jnp.where(mask1, w1r_ref[...], 0.0)).astype(jnp.bfloat16)
            out_ref[...] += jax.lax.dot_general(
                oh_w, ys_ref[...].astype(jnp.bfloat16),
                (((0,), (0,)), ((), ())),
                preferred_element_type=jnp.float32)


def _route_body(ns, rw_ref, x_ref, rb_ref, d0_ref, d1_ref, w0_ref, w1_ref,
                be_ref, na_ref):
    T, E, BLK, NB = ns
    lt = jax.lax.dot_general(
        rw_ref[...], x_ref[...], (((0,), (1,)), ((), ())),
        preferred_element_type=jnp.float32) + rb_ref[...]
    e_col = jax.lax.broadcasted_iota(jnp.int32, (E, 1), 0)
    l1 = jnp.max(lt, axis=0, keepdims=True)
    i1 = jnp.min(jnp.where(lt == l1, e_col, E), axis=0,
                 keepdims=True).astype(jnp.int32)
    masked = jnp.where(i1 == e_col, -jnp.inf, lt)
    l2 = jnp.max(masked, axis=0, keepdims=True)
    i2 = jnp.min(jnp.where(masked == l2, e_col, E), axis=0,
                 keepdims=True).astype(jnp.int32)
    w0_ref[...] = jax.nn.sigmoid(l1 - l2)
    w1_ref[...] = jax.nn.sigmoid(l2 - l1)

    e_row = jnp.concatenate([i1, i2], axis=1)
    onehot = (e_row == e_col).astype(jnp.float32)
    cum = onehot
    s = 1
    while s < 2 * T:
        cum = cum + jnp.pad(cum[:, :-s], ((0, 0), (s, 0)))
        s *= 2
    rank = jnp.sum(cum * onehot, axis=0, keepdims=True) - 1.0
    counts = jnp.sum(onehot, axis=1, keepdims=True)
    nblk = jnp.floor((counts + (BLK - 1)) * (1.0 / BLK))
    r_iota = jax.lax.broadcasted_iota(jnp.int32, (E, E), 0)
    c_iota = jax.lax.broadcasted_iota(jnp.int32, (E, E), 1)
    tril_s = (r_iota > c_iota).astype(jnp.float32)
    tril_i = (r_iota >= c_iota).astype(jnp.float32)
    blk_start = jnp.dot(tril_s, nblk,
                        preferred_element_type=jnp.float32)
    bounds = jnp.dot(tril_i, nblk,
                     preferred_element_type=jnp.float32)
    row_start = blk_start * float(BLK)
    dest = jnp.sum(onehot * row_start, axis=0, keepdims=True) + rank
    d0_ref[...] = dest[:, :T].astype(jnp.int32)
    d1_ref[...] = dest[:, T:].astype(jnp.int32)

    na = jnp.sum(nblk)
    ba = jax.lax.broadcasted_iota(jnp.int32, (1, NB), 1).astype(jnp.float32)
    be_raw = jnp.minimum(
        jnp.sum((ba >= bounds).astype(jnp.float32), axis=0, keepdims=True),
        float(E - 1))
    be_last = jnp.sum(jnp.where(ba == na - 1.0, be_raw, 0.0), axis=1,
                      keepdims=True)
    be_ref[...] = jnp.where(ba < na, be_raw, be_last).astype(jnp.int32)
    na_ref[...] = jnp.full((1, 1), na, jnp.float32).astype(jnp.int32)


def kernel(hidden_states, router_w, router_b, fc1_w, fc1_b, fc2_w, fc2_b):
    BATCH, S, D = hidden_states.shape
    E = router_w.shape[1]
    F = fc1_w.shape[2]
    K = 2
    T = BATCH * S
    BLK = 512 if T % 512 == 0 else 8
    FT = 512 if F % 512 == 0 else F
    NF = F // FT
    NB = (T * K) // BLK + E

    x = hidden_states.reshape(T, D).astype(jnp.bfloat16)

    _r1 = lambda i: (0, 0)
    d0, d1, w0, w1r, be2, na2 = pl.pallas_call(
        functools.partial(_route_body, (T, E, BLK, NB)),
        grid=(1,),
        in_specs=[
            pl.BlockSpec((D, E), _r1),
            pl.BlockSpec((T, D), _r1),
            pl.BlockSpec((E, 1), _r1),
        ],
        out_specs=[
            pl.BlockSpec((1, T), _r1),
            pl.BlockSpec((1, T), _r1),
            pl.BlockSpec((1, T), _r1),
            pl.BlockSpec((1, T), _r1),
            pl.BlockSpec((1, NB), _r1),
            pl.BlockSpec((1, 1), _r1),
        ],
        out_shape=[
            jax.ShapeDtypeStruct((1, T), jnp.int32),
            jax.ShapeDtypeStruct((1, T), jnp.int32),
            jax.ShapeDtypeStruct((1, T), jnp.float32),
            jax.ShapeDtypeStruct((1, T), jnp.float32),
            jax.ShapeDtypeStruct((1, NB), jnp.int32),
            jax.ShapeDtypeStruct((1, 1), jnp.int32),
        ],
    )(router_w.astype(jnp.bfloat16), x, router_b.reshape(E, 1))
    block_expert = be2.reshape(NB)
    na = na2.reshape(1)

    fc1_b3 = fc1_b.reshape(E, 1, F)
    fc2_b3 = fc2_b.reshape(E, 1, D)

    def wmap(b, f, be, na_s):
        ff = jnp.where(b < na_s[0], f, NF - 1)
        return (be[b], 0, ff)

    _res = lambda b, f, be, na_s: (0, 0)
    grid_spec = pltpu.PrefetchScalarGridSpec(
        num_scalar_prefetch=2,
        grid=(NB, NF),
        in_specs=[
            pl.BlockSpec((T, D), _res),
            pl.BlockSpec((1, T), _res),
            pl.BlockSpec((1, T), _res),
            pl.BlockSpec((1, T), _res),
            pl.BlockSpec((1, T), _res),
            pl.BlockSpec((1, D, FT), wmap),
            pl.BlockSpec((1, 1, FT),
                         lambda b, f, be, na_s:
                         (be[b], 0, jnp.where(b < na_s[0], f, NF - 1))),
            pl.BlockSpec((1, FT, D),
                         lambda b, f, be, na_s:
                         (be[b], jnp.where(b < na_s[0], f, NF - 1), 0)),
            pl.BlockSpec((1, 1, D), lambda b, f, be, na_s: (be[b], 0, 0)),
        ],
        out_specs=pl.BlockSpec((T, D), _res),
        scratch_shapes=[
            pltpu.VMEM((BLK, D), jnp.bfloat16),
            pltpu.VMEM((BLK, D), jnp.float32),
        ],
    )

    out = pl.pallas_call(
        functools.partial(_moe_body, (BLK, T, NF)),
        grid_spec=grid_spec,
        out_shape=jax.ShapeDtypeStruct((T, D), jnp.float32),
        compiler_params=pltpu.CompilerParams(
            dimension_semantics=("arbitrary", "arbitrary")),
    )(block_expert, na,
      x, d0, d1, w0, w1r, fc1_w, fc1_b3, fc2_w, fc2_b3)

    return out.reshape(BATCH, S, D)


# speedup vs baseline: 1.5369x; 1.4009x over previous
import functools

import jax
import jax.numpy as jnp
from jax.experimental import pallas as pl
from jax.experimental.pallas import tpu as pltpu


def _moe_body(ns, be_ref, na_ref, x_ref, d0_ref, d1_ref, w0_ref, w1r_ref,
              w1_ref, b1_ref, w2_ref, b2_ref, out_ref, xs_ref, ys_ref):
    BLK, S, NF = ns
    b = pl.program_id(0)
    f = pl.program_id(1)
    active = b < na_ref[0]

    slot = b * BLK + jax.lax.broadcasted_iota(jnp.int32, (BLK, 1), 0)

    @pl.when(active & (f == 0))
    def _gather():
        mask0 = slot == d0_ref[...]
        mask1 = slot == d1_ref[...]
        onehot = (mask0 | mask1).astype(jnp.bfloat16)
        xs_ref[...] = jnp.dot(onehot, x_ref[...],
                              preferred_element_type=jnp.float32
                              ).astype(jnp.bfloat16)
        ys_ref[...] = jnp.broadcast_to(b2_ref[0], ys_ref.shape)

    @pl.when(active)
    def _ffn():
        h = jnp.dot(xs_ref[...], w1_ref[0].astype(jnp.bfloat16),
                    preferred_element_type=jnp.float32) + b1_ref[0]
        h = jax.nn.gelu(h, approximate=True)
        ys_ref[...] += jnp.dot(h.astype(jnp.bfloat16),
                               w2_ref[0].astype(jnp.bfloat16),
                               preferred_element_type=jnp.float32)

    @pl.when(f == NF - 1)
    def _scatter():
        @pl.when(b == 0)
        def _init():
            out_ref[...] = jnp.zeros_like(out_ref)

        @pl.when(active)
        def _add():
            mask0 = slot == d0_ref[...]
            mask1 = slot == d1_ref[...]
            oh_w = (jnp.where(mask0, w0_ref[...], 0.0)
                    + jnp.where(mask1, w1r_ref[...], 0.0)).astype(jnp.bfloat16)
            out_ref[...] += jax.lax.dot_general(
                oh_w, ys_ref[...].astype(jnp.bfloat16),
                (((0,), (0,)), ((), ())),
                preferred_element_type=jnp.float32)


def _route_body(ns, rw_ref, x_ref, rb_ref, d0_ref, d1_ref, w0_ref, w1_ref,
                be_ref, na_ref):
    T, E, BLK, NB = ns
    lt = jax.lax.dot_general(
        rw_ref[...], x_ref[...], (((0,), (1,)), ((), ())),
        preferred_element_type=jnp.float32) + rb_ref[...]
    e_col = jax.lax.broadcasted_iota(jnp.int32, (E, 1), 0)
    l1 = jnp.max(lt, axis=0, keepdims=True)
    i1 = jnp.min(jnp.where(lt == l1, e_col, E), axis=0,
                 keepdims=True).astype(jnp.int32)
    masked = jnp.where(i1 == e_col, -jnp.inf, lt)
    l2 = jnp.max(masked, axis=0, keepdims=True)
    i2 = jnp.min(jnp.where(masked == l2, e_col, E), axis=0,
                 keepdims=True).astype(jnp.int32)
    w0_ref[...] = jax.nn.sigmoid(l1 - l2)
    w1_ref[...] = jax.nn.sigmoid(l2 - l1)

    e_row = jnp.concatenate([i1, i2], axis=1)
    onehot = (e_row == e_col).astype(jnp.float32)
    cum = onehot
    s = 1
    while s < 2 * T:
        cum = cum + jnp.pad(cum[:, :-s], ((0, 0), (s, 0)))
        s *= 2
    rank = jnp.sum(cum * onehot, axis=0, keepdims=True) - 1.0
    counts = jnp.sum(onehot, axis=1, keepdims=True)
    nblk = jnp.floor((counts + (BLK - 1)) * (1.0 / BLK))
    r_iota = jax.lax.broadcasted_iota(jnp.int32, (E, E), 0)
    c_iota = jax.lax.broadcasted_iota(jnp.int32, (E, E), 1)
    tril_s = (r_iota > c_iota).astype(jnp.float32)
    tril_i = (r_iota >= c_iota).astype(jnp.float32)
    blk_start = jnp.dot(tril_s, nblk,
                        preferred_element_type=jnp.float32)
    bounds = jnp.dot(tril_i, nblk,
                     preferred_element_type=jnp.float32)
    row_start = blk_start * float(BLK)
    dest = jnp.sum(onehot * row_start, axis=0, keepdims=True) + rank
    d0_ref[...] = dest[:, :T].astype(jnp.int32)
    d1_ref[...] = dest[:, T:].astype(jnp.int32)

    na = jnp.sum(nblk)
    ba = jax.lax.broadcasted_iota(jnp.int32, (1, NB), 1).astype(jnp.float32)
    be_raw = jnp.minimum(
        jnp.sum((ba >= bounds).astype(jnp.float32), axis=0, keepdims=True),
        float(E - 1))
    be_last = jnp.sum(jnp.where(ba == na - 1.0, be_raw, 0.0), axis=1,
                      keepdims=True)
    be_ref[...] = jnp.where(ba < na, be_raw, be_last).astype(jnp.int32)
    na_ref[...] = jnp.full((1, 1), na, jnp.float32).astype(jnp.int32)


def kernel(hidden_states, router_w, router_b, fc1_w, fc1_b, fc2_w, fc2_b):
    BATCH, S, D = hidden_states.shape
    E = router_w.shape[1]
    F = fc1_w.shape[2]
    K = 2
    T = BATCH * S
    BLK = 576 if T >= 2048 else 8
    FT = 512 if F % 512 == 0 else F
    NF = F // FT
    NB = (T * K) // BLK + E

    x = hidden_states.reshape(T, D).astype(jnp.bfloat16)

    _r1 = lambda i: (0, 0)
    d0, d1, w0, w1r, be2, na2 = pl.pallas_call(
        functools.partial(_route_body, (T, E, BLK, NB)),
        grid=(1,),
        in_specs=[
            pl.BlockSpec((D, E), _r1),
            pl.BlockSpec((T, D), _r1),
            pl.BlockSpec((E, 1), _r1),
        ],
        out_specs=[
            pl.BlockSpec((1, T), _r1),
            pl.BlockSpec((1, T), _r1),
            pl.BlockSpec((1, T), _r1),
            pl.BlockSpec((1, T), _r1),
            pl.BlockSpec((1, NB), _r1),
            pl.BlockSpec((1, 1), _r1),
        ],
        out_shape=[
            jax.ShapeDtypeStruct((1, T), jnp.int32),
            jax.ShapeDtypeStruct((1, T), jnp.int32),
            jax.ShapeDtypeStruct((1, T), jnp.float32),
            jax.ShapeDtypeStruct((1, T), jnp.float32),
            jax.ShapeDtypeStruct((1, NB), jnp.int32),
            jax.ShapeDtypeStruct((1, 1), jnp.int32),
        ],
    )(router_w.astype(jnp.bfloat16), x, router_b.reshape(E, 1))
    block_expert = be2.reshape(NB)
    na = na2.reshape(1)

    fc1_b3 = fc1_b.reshape(E, 1, F)
    fc2_b3 = fc2_b.reshape(E, 1, D)

    def wmap(b, f, be, na_s):
        ff = jnp.where(b < na_s[0], f, NF - 1)
        return (be[b], 0, ff)

    _res = lambda b, f, be, na_s: (0, 0)
    grid_spec = pltpu.PrefetchScalarGridSpec(
        num_scalar_prefetch=2,
        grid=(NB, NF),
        in_specs=[
            pl.BlockSpec((T, D), _res),
            pl.BlockSpec((1, T), _res),
            pl.BlockSpec((1, T), _res),
            pl.BlockSpec((1, T), _res),
            pl.BlockSpec((1, T), _res),
            pl.BlockSpec((1, D, FT), wmap),
            pl.BlockSpec((1, 1, FT),
                         lambda b, f, be, na_s:
                         (be[b], 0, jnp.where(b < na_s[0], f, NF - 1))),
            pl.BlockSpec((1, FT, D),
                         lambda b, f, be, na_s:
                         (be[b], jnp.where(b < na_s[0], f, NF - 1), 0)),
            pl.BlockSpec((1, 1, D), lambda b, f, be, na_s: (be[b], 0, 0)),
        ],
        out_specs=pl.BlockSpec((T, D), _res),
        scratch_shapes=[
            pltpu.VMEM((BLK, D), jnp.bfloat16),
            pltpu.VMEM((BLK, D), jnp.float32),
        ],
    )

    out = pl.pallas_call(
        functools.partial(_moe_body, (BLK, T, NF)),
        grid_spec=grid_spec,
        out_shape=jax.ShapeDtypeStruct((T, D), jnp.float32),
        compiler_params=pltpu.CompilerParams(
            dimension_semantics=("arbitrary", "arbitrary")),
    )(block_expert, na,
      x, d0, d1, w0, w1r, fc1_w, fc1_b3, fc2_w, fc2_b3)

    return out.reshape(BATCH, S, D)
